# Initial kernel scaffold; baseline (speedup 1.0000x reference)
import jax
import jax.numpy as jnp
from jax.experimental import pallas as pl


def kernel(a, b):
    raise NotImplementedError("write your pallas kernel here")


# speedup vs baseline: 40.8958x; 40.8958x over previous
import jax
import jax.numpy as jnp
from jax.experimental import pallas as pl
from jax.experimental.pallas import tpu as pltpu

_LANES = 512


def _ciou_block(px_ref, py_ref, out_ref):
    px = px_ref[...]
    py = py_ref[...]

    ax, bx = px[0:8, :], px[8:16, :]
    ay, by = py[0:8, :], py[8:16, :]

    def nxt(v):
        return jnp.concatenate([v[1:8, :], v[0:1, :]], axis=0)

    dax, day = nxt(ax) - ax, nxt(ay) - ay
    dbx, dby = nxt(bx) - bx, nxt(by) - by

    area_a = 0.5 * jnp.sum(ax * day - ay * dax, axis=0, keepdims=True)
    area_b = 0.5 * jnp.sum(bx * dby - by * dbx, axis=0, keepdims=True)

    def clip(sx, sy, dx, dy, hx, hy, hdx, hdy):
        t1 = jnp.zeros_like(sx)
        t2 = jnp.ones_like(sx)
        dead = jnp.zeros(sx.shape, dtype=jnp.bool_)
        for j in range(8):
            ex = hdx[j:j + 1, :]
            ey = hdy[j:j + 1, :]
            c0 = ex * (sy - hy[j:j + 1, :]) - ey * (sx - hx[j:j + 1, :])
            cd = ex * dy - ey * dx
            para = cd == 0.0
            r = -c0 / jnp.where(para, 1.0, cd)
            t1 = jnp.maximum(t1, jnp.where(cd > 0.0, r, 0.0))
            t2 = jnp.minimum(t2, jnp.where(cd < 0.0, r, 1.0))
            dead = dead | (para & (c0 < 0.0))
        keep = (t2 > t1) & ~dead
        return jnp.where(keep, (t2 - t1) * (sx * dy - sy * dx), 0.0)

    inter = 0.5 * (
        jnp.sum(clip(ax, ay, dax, day, bx, by, dbx, dby), axis=0, keepdims=True)
        + jnp.sum(clip(bx, by, dbx, dby, ax, ay, dax, day), axis=0,
                  keepdims=True))

    rows = jax.lax.broadcasted_iota(jnp.int32, px.shape, 0)
    acc = jnp.zeros_like(area_a)
    for i in range(16):
        pix = px[i:i + 1, :]
        piy = py[i:i + 1, :]
        ex = px - pix
        ey = py - piy
        mincr = jnp.zeros_like(px)
        for k in range(16):
            if k == i:
                continue
            wx = px[k:k + 1, :] - pix
            wy = py[k:k + 1, :] - piy
            cr = jnp.where(rows == k, 0.0, ex * wy - ey * wx)
            mincr = jnp.minimum(mincr, cr)
        contrib = jnp.where(mincr >= 0.0, pix * py - piy * px, 0.0)
        acc = acc + jnp.sum(contrib, axis=0, keepdims=True)
    ch_area = 0.5 * acc

    union = area_a + area_b - inter
    iou = inter / union
    ciou = iou - (ch_area - union) / ch_area

    part = ciou[:, 0:128]
    for c in range(1, _LANES // 128):
        part = part + ciou[:, c * 128:(c + 1) * 128]
    out_ref[0, :, :] = part


def kernel(a, b):
    bsz = a.shape[0]
    pts = jnp.concatenate([a, b], axis=1)
    px = pts[..., 0].T
    py = pts[..., 1].T
    g = bsz // _LANES
    partials = pl.pallas_call(
        _ciou_block,
        grid=(g,),
        in_specs=[pl.BlockSpec((16, _LANES), lambda i: (0, i)),
                  pl.BlockSpec((16, _LANES), lambda i: (0, i))],
        out_specs=pl.BlockSpec((1, 1, 128), lambda i: (i, 0, 0)),
        out_shape=jax.ShapeDtypeStruct((g, 1, 128), jnp.float32),
        compiler_params=pltpu.CompilerParams(
            dimension_semantics=("parallel",)),
        name="ciou_mean",
    )(px, py)
    return jnp.sum(partials) / bsz
